# Initial kernel scaffold; baseline (speedup 1.0000x reference)
import jax
import jax.numpy as jnp
from jax import lax
from jax.experimental import pallas as pl
from jax.experimental.pallas import tpu as pltpu
from jax.experimental.pallas import tpu_sc as plsc

BATCH = 16384
HID = 64
NC = 2
NS = 16
NW = NC * NS
B_PER_W = BATCH // NW
WIN = 128
NWIN = B_PER_W // WIN


def _sc_gather_body(pt_hbm, nt_hbm, pidx_hbm, nidx_hbm, hp_hbm, hn_hbm,
                    pidx_v, nidx_v, pv, nv, psem, nsem):
    wid = lax.axis_index("s") * NC + lax.axis_index("c")
    base = wid * B_PER_W
    row0 = wid * NWIN
    pltpu.sync_copy(pidx_hbm.at[pl.ds(row0, NWIN)], pidx_v)
    pltpu.sync_copy(nidx_hbm.at[pl.ds(row0, NWIN)], nidx_v)
    copies = []
    for k in range(NWIN):
        copies.append(pltpu.async_copy(
            pt_hbm.at[pidx_v.at[k]], pv.at[pl.ds(k * WIN, WIN)], psem))
        copies.append(pltpu.async_copy(
            nt_hbm.at[nidx_v.at[k]], nv.at[pl.ds(k * WIN, WIN)], nsem))
    for c in copies:
        c.wait()
    pltpu.sync_copy(pv, hp_hbm.at[pl.ds(base, B_PER_W)])
    pltpu.sync_copy(nv, hn_hbm.at[pl.ds(base, B_PER_W)])


def _sc_gather(pt, nt, pidx, nidx):
    mesh = plsc.VectorSubcoreMesh(core_axis_name="c", subcore_axis_name="s")
    f = pl.kernel(
        _sc_gather_body,
        out_type=(jax.ShapeDtypeStruct((BATCH, HID), jnp.float32),
                  jax.ShapeDtypeStruct((BATCH, HID), jnp.float32)),
        mesh=mesh,
        scratch_types=[
            pltpu.VMEM((NWIN, WIN), jnp.int32),
            pltpu.VMEM((NWIN, WIN), jnp.int32),
            pltpu.VMEM((B_PER_W, HID), jnp.float32),
            pltpu.VMEM((B_PER_W, HID), jnp.float32),
            pltpu.SemaphoreType.DMA,
            pltpu.SemaphoreType.DMA,
        ],
    )
    return f(pt, nt, pidx, nidx)


def _mlp_body(hp_ref, hn_ref, w1a_ref, w1b_ref, b1_ref, w2_ref, b2_ref, out_ref):
    z = jnp.dot(hp_ref[...], w1a_ref[...], preferred_element_type=jnp.float32)
    z = z + jnp.dot(hn_ref[...], w1b_ref[...], preferred_element_type=jnp.float32)
    z = jnp.maximum(z + b1_ref[...], 0.0)
    out_ref[...] = jnp.sum(z * w2_ref[...], axis=1, keepdims=True) + b2_ref[...]


def _mlp(hp, hn, w1a, w1b, b1, w2row, b2, block_rows=2048):
    grid = (BATCH // block_rows,)
    return pl.pallas_call(
        _mlp_body,
        grid=grid,
        in_specs=[
            pl.BlockSpec((block_rows, HID), lambda i: (i, 0)),
            pl.BlockSpec((block_rows, HID), lambda i: (i, 0)),
            pl.BlockSpec((HID, 16), lambda i: (0, 0)),
            pl.BlockSpec((HID, 16), lambda i: (0, 0)),
            pl.BlockSpec((1, 16), lambda i: (0, 0)),
            pl.BlockSpec((1, 16), lambda i: (0, 0)),
            pl.BlockSpec((1, 1), lambda i: (0, 0)),
        ],
        out_specs=pl.BlockSpec((block_rows, 1), lambda i: (i, 0)),
        out_shape=jax.ShapeDtypeStruct((BATCH, 1), jnp.float32),
    )(hp, hn, w1a, w1b, b1, w2row, b2)


@jax.jit
def kernel(x, emb_proton, emb_neutron, W1, b1, W2, b2):
    pidx = x[:, 0].reshape(NW * NWIN, WIN)
    nidx = x[:, 1].reshape(NW * NWIN, WIN)
    hp, hn = _sc_gather(emb_proton, emb_neutron, pidx, nidx)
    return _mlp(hp, hn, W1[:HID], W1[HID:], b1.reshape(1, 16),
                W2.reshape(1, 16), b2.reshape(1, 1))


# speedup vs baseline: 1.1621x; 1.1621x over previous
import jax
import jax.numpy as jnp
from jax import lax
from jax.experimental import pallas as pl
from jax.experimental.pallas import tpu as pltpu
from jax.experimental.pallas import tpu_sc as plsc

BATCH = 16384
HID = 64
NC = 2
NS = 16
NW = NC * NS
B_PER_W = BATCH // NW
WIN = 128
NWIN = B_PER_W // WIN


def _sc_gather_body(pt_hbm, nt_hbm, pidx_hbm, nidx_hbm, hp_hbm, hn_hbm,
                    pidx_v, nidx_v, pv, nv, psem, nsem):
    wid = lax.axis_index("s") * NC + lax.axis_index("c")
    base = wid * B_PER_W
    row0 = wid * NWIN
    pltpu.sync_copy(pidx_hbm.at[pl.ds(row0, NWIN)], pidx_v)
    pltpu.sync_copy(nidx_hbm.at[pl.ds(row0, NWIN)], nidx_v)
    copies = []
    for k in range(NWIN):
        copies.append(pltpu.async_copy(
            pt_hbm.at[pidx_v.at[k]], pv.at[pl.ds(k * WIN, WIN)], psem))
        copies.append(pltpu.async_copy(
            nt_hbm.at[nidx_v.at[k]], nv.at[pl.ds(k * WIN, WIN)], nsem))
    for c in copies:
        c.wait()
    pltpu.sync_copy(pv, hp_hbm.at[pl.ds(base, B_PER_W)])
    pltpu.sync_copy(nv, hn_hbm.at[pl.ds(base, B_PER_W)])


def _sc_gather(pt, nt, pidx, nidx):
    mesh = plsc.VectorSubcoreMesh(core_axis_name="c", subcore_axis_name="s")
    f = pl.kernel(
        _sc_gather_body,
        out_type=(jax.ShapeDtypeStruct((BATCH, HID), jnp.float32),
                  jax.ShapeDtypeStruct((BATCH, HID), jnp.float32)),
        mesh=mesh,
        compiler_params=pltpu.CompilerParams(use_tc_tiling_on_sc=False),
        scratch_types=[
            pltpu.VMEM((NWIN, WIN), jnp.int32),
            pltpu.VMEM((NWIN, WIN), jnp.int32),
            pltpu.VMEM((B_PER_W, HID), jnp.float32),
            pltpu.VMEM((B_PER_W, HID), jnp.float32),
            pltpu.SemaphoreType.DMA,
            pltpu.SemaphoreType.DMA,
        ],
    )
    return f(pt, nt, pidx, nidx)


def _mlp_body(hp_ref, hn_ref, w1a_ref, w1b_ref, b1_ref, w2_ref, b2_ref, out_ref):
    z = jnp.dot(hp_ref[...], w1a_ref[...], preferred_element_type=jnp.float32)
    z = z + jnp.dot(hn_ref[...], w1b_ref[...], preferred_element_type=jnp.float32)
    z = jnp.maximum(z + b1_ref[...], 0.0)
    out_ref[...] = jnp.sum(z * w2_ref[...], axis=1, keepdims=True) + b2_ref[...]


def _mlp(hp, hn, w1a, w1b, b1, w2row, b2, block_rows=2048):
    grid = (BATCH // block_rows,)
    return pl.pallas_call(
        _mlp_body,
        grid=grid,
        in_specs=[
            pl.BlockSpec((block_rows, HID), lambda i: (i, 0)),
            pl.BlockSpec((block_rows, HID), lambda i: (i, 0)),
            pl.BlockSpec((HID, 16), lambda i: (0, 0)),
            pl.BlockSpec((HID, 16), lambda i: (0, 0)),
            pl.BlockSpec((1, 16), lambda i: (0, 0)),
            pl.BlockSpec((1, 16), lambda i: (0, 0)),
            pl.BlockSpec((1, 1), lambda i: (0, 0)),
        ],
        out_specs=pl.BlockSpec((block_rows, 1), lambda i: (i, 0)),
        out_shape=jax.ShapeDtypeStruct((BATCH, 1), jnp.float32),
    )(hp, hn, w1a, w1b, b1, w2row, b2)


@jax.jit
def kernel(x, emb_proton, emb_neutron, W1, b1, W2, b2):
    pidx = x[:, 0].reshape(NW * NWIN, WIN)
    nidx = x[:, 1].reshape(NW * NWIN, WIN)
    hp, hn = _sc_gather(emb_proton, emb_neutron, pidx, nidx)
    return _mlp(hp, hn, W1[:HID], W1[HID:], b1.reshape(1, 16),
                W2.reshape(1, 16), b2.reshape(1, 1))


# speedup vs baseline: 1.4241x; 1.2254x over previous
import jax
import jax.numpy as jnp
from jax import lax
from jax.experimental import pallas as pl
from jax.experimental.pallas import tpu as pltpu
from jax.experimental.pallas import tpu_sc as plsc

BATCH = 16384
HID = 64
NC = 2
NS = 16
NW = NC * NS
B_PER_W = BATCH // NW
CHUNK = 128
K = 16


def _sc_gather_body(pt_hbm, nt_hbm, pidx_hbm, nidx_hbm, hp_hbm, hn_hbm,
                    pidx_v, nidx_v, pv, nv, psem, nsem):
    wid = lax.axis_index("s") * NC + lax.axis_index("c")
    base = wid * B_PER_W
    pltpu.sync_copy(pidx_hbm.at[pl.ds(base, B_PER_W)], pidx_v)
    pltpu.sync_copy(nidx_hbm.at[pl.ds(base, B_PER_W)], nidx_v)

    @pl.loop(0, B_PER_W // CHUNK)
    def _chunk(c):
        @pl.loop(0, CHUNK // K)
        def _win(w):
            pvec = pidx_v[pl.ds(c * CHUNK + w * K, K)]
            nvec = nidx_v[pl.ds(c * CHUNK + w * K, K)]
            copies = []
            for j in range(K):
                d = w * K + j
                copies.append(pltpu.async_copy(
                    pt_hbm.at[pvec[j]], pv.at[d], psem))
                copies.append(pltpu.async_copy(
                    nt_hbm.at[nvec[j]], nv.at[d], nsem))
            for cp in copies:
                cp.wait()
        pltpu.sync_copy(pv, hp_hbm.at[pl.ds(base + c * CHUNK, CHUNK)])
        pltpu.sync_copy(nv, hn_hbm.at[pl.ds(base + c * CHUNK, CHUNK)])


def _sc_gather(pt, nt, pidx, nidx):
    mesh = plsc.VectorSubcoreMesh(core_axis_name="c", subcore_axis_name="s")
    f = pl.kernel(
        _sc_gather_body,
        out_type=(jax.ShapeDtypeStruct((BATCH, HID), jnp.float32),
                  jax.ShapeDtypeStruct((BATCH, HID), jnp.float32)),
        mesh=mesh,
        scratch_types=[
            pltpu.VMEM((B_PER_W,), jnp.int32),
            pltpu.VMEM((B_PER_W,), jnp.int32),
            pltpu.VMEM((CHUNK, HID), jnp.float32),
            pltpu.VMEM((CHUNK, HID), jnp.float32),
            pltpu.SemaphoreType.DMA,
            pltpu.SemaphoreType.DMA,
        ],
    )
    return f(pt, nt, pidx, nidx)


def _mlp_body(hp_ref, hn_ref, w1a_ref, w1b_ref, b1_ref, w2_ref, b2_ref, out_ref):
    z = jnp.dot(hp_ref[...], w1a_ref[...], preferred_element_type=jnp.float32)
    z = z + jnp.dot(hn_ref[...], w1b_ref[...], preferred_element_type=jnp.float32)
    z = jnp.maximum(z + b1_ref[...], 0.0)
    out_ref[...] = jnp.sum(z * w2_ref[...], axis=1, keepdims=True) + b2_ref[...]


def _mlp(hp, hn, w1a, w1b, b1, w2row, b2, block_rows=2048):
    grid = (BATCH // block_rows,)
    return pl.pallas_call(
        _mlp_body,
        grid=grid,
        in_specs=[
            pl.BlockSpec((block_rows, HID), lambda i: (i, 0)),
            pl.BlockSpec((block_rows, HID), lambda i: (i, 0)),
            pl.BlockSpec((HID, 16), lambda i: (0, 0)),
            pl.BlockSpec((HID, 16), lambda i: (0, 0)),
            pl.BlockSpec((1, 16), lambda i: (0, 0)),
            pl.BlockSpec((1, 16), lambda i: (0, 0)),
            pl.BlockSpec((1, 1), lambda i: (0, 0)),
        ],
        out_specs=pl.BlockSpec((block_rows, 1), lambda i: (i, 0)),
        out_shape=jax.ShapeDtypeStruct((BATCH, 1), jnp.float32),
    )(hp, hn, w1a, w1b, b1, w2row, b2)


@jax.jit
def kernel(x, emb_proton, emb_neutron, W1, b1, W2, b2):
    pidx = x[:, 0]
    nidx = x[:, 1]
    hp, hn = _sc_gather(emb_proton, emb_neutron, pidx, nidx)
    return _mlp(hp, hn, W1[:HID], W1[HID:], b1.reshape(1, 16),
                W2.reshape(1, 16), b2.reshape(1, 1))


# speedup vs baseline: 1.4247x; 1.0004x over previous
import jax
import jax.numpy as jnp
from jax import lax
from jax.experimental import pallas as pl
from jax.experimental.pallas import tpu as pltpu
from jax.experimental.pallas import tpu_sc as plsc

BATCH = 16384
HID = 64
NC = 2
NS = 16
NW = NC * NS
B_PER_W = BATCH // NW
CHUNK = 128
NCHUNK = B_PER_W // CHUNK
K = 16


def _sc_gather_body(pt_hbm, nt_hbm, x_hbm, h_hbm, xv, hv0, hv1, gsem, wsem):
    wid = lax.axis_index("s") * NC + lax.axis_index("c")
    base = wid * B_PER_W
    pltpu.sync_copy(x_hbm.at[pl.ds(base, B_PER_W)], xv)

    iota = lax.iota(jnp.int32, 16)
    zeros = jnp.zeros((16,), jnp.int32)
    ones = jnp.ones((16,), jnp.int32)

    hvs = (hv0, hv1)
    writeouts = []
    for c in range(NCHUNK):
        hv = hvs[c % 2]
        if c >= 2:
            writeouts[c - 2].wait()

        @pl.loop(0, CHUNK // K)
        def _win(w, c=c, hv=hv):
            r0 = c * CHUNK + w * K
            pvec = plsc.load_gather(xv, [r0 + iota, zeros])
            nvec = plsc.load_gather(xv, [r0 + iota, ones])
            copies = []
            for j in range(K):
                d = w * K + j
                copies.append(pltpu.async_copy(
                    pt_hbm.at[pvec[j]], hv.at[d, pl.ds(0, HID)], gsem))
                copies.append(pltpu.async_copy(
                    nt_hbm.at[nvec[j]], hv.at[d, pl.ds(HID, HID)], gsem))
            for cp in copies:
                cp.wait()

        writeouts.append(pltpu.async_copy(
            hv, h_hbm.at[pl.ds(base + c * CHUNK, CHUNK)], wsem))
    for wo in writeouts[-2:]:
        wo.wait()


def _sc_gather(pt, nt, x):
    mesh = plsc.VectorSubcoreMesh(core_axis_name="c", subcore_axis_name="s")
    f = pl.kernel(
        _sc_gather_body,
        out_type=jax.ShapeDtypeStruct((BATCH, 2 * HID), jnp.float32),
        mesh=mesh,
        compiler_params=pltpu.CompilerParams(needs_layout_passes=False),
        scratch_types=[
            pltpu.VMEM((B_PER_W, 2), jnp.int32),
            pltpu.VMEM((CHUNK, 2 * HID), jnp.float32),
            pltpu.VMEM((CHUNK, 2 * HID), jnp.float32),
            pltpu.SemaphoreType.DMA,
            pltpu.SemaphoreType.DMA,
        ],
    )
    return f(pt, nt, x)


def _mlp_body(h_ref, w1_ref, b1_ref, w2_ref, b2_ref, out_ref):
    z = jnp.dot(h_ref[...], w1_ref[...], preferred_element_type=jnp.float32)
    z = jnp.maximum(z + b1_ref[...], 0.0)
    out_ref[...] = jnp.sum(z * w2_ref[...], axis=1, keepdims=True) + b2_ref[...]


def _mlp(h, w1, b1, w2row, b2, block_rows=2048):
    grid = (BATCH // block_rows,)
    return pl.pallas_call(
        _mlp_body,
        grid=grid,
        in_specs=[
            pl.BlockSpec((block_rows, 2 * HID), lambda i: (i, 0)),
            pl.BlockSpec((2 * HID, 16), lambda i: (0, 0)),
            pl.BlockSpec((1, 16), lambda i: (0, 0)),
            pl.BlockSpec((1, 16), lambda i: (0, 0)),
            pl.BlockSpec((1, 1), lambda i: (0, 0)),
        ],
        out_specs=pl.BlockSpec((block_rows, 1), lambda i: (i, 0)),
        out_shape=jax.ShapeDtypeStruct((BATCH, 1), jnp.float32),
    )(h, w1, b1, w2row, b2)


@jax.jit
def kernel(x, emb_proton, emb_neutron, W1, b1, W2, b2):
    h = _sc_gather(emb_proton, emb_neutron, x)
    return _mlp(h, W1, b1.reshape(1, 16), W2.reshape(1, 16), b2.reshape(1, 1))


# speedup vs baseline: 2.2620x; 1.5877x over previous
import jax
import jax.numpy as jnp
from jax import lax
from jax.experimental import pallas as pl
from jax.experimental.pallas import tpu as pltpu
from jax.experimental.pallas import tpu_sc as plsc

BATCH = 16384
HID = 64
NC = 2
NS = 16
DPW = HID // NS
VOC = 100000
OCHUNK = 4096
NOC = BATCH // OCHUNK


def _sc_gather_body(ptT_hbm, ntT_hbm, xT_hbm, hT_hbm,
                    idxv, rowv, ob0, ob1, osem):
    cid = lax.axis_index("c")
    sid = lax.axis_index("s")
    wid = sid * NC + cid
    obs = (ob0, ob1)

    for phase, tbl in ((0, ptT_hbm), (1, ntT_hbm)):
        pltpu.sync_copy(xT_hbm.at[phase], idxv)
        for k in range(2):
            d = wid * 2 + k
            pltpu.sync_copy(tbl.at[d], rowv)
            g = phase * HID + d
            writeouts = []
            for ci in range(NOC):
                ob = obs[ci % 2]
                if ci >= 2:
                    writeouts[ci - 2].wait()

                @pl.loop(0, OCHUNK // 16)
                def _gather(t, ci=ci, ob=ob):
                    iv = idxv[pl.ds(ci * OCHUNK + t * 16, 16)]
                    ob[pl.ds(t * 16, 16)] = plsc.load_gather(rowv, [iv])

                writeouts.append(pltpu.async_copy(
                    ob, hT_hbm.at[g, pl.ds(ci * OCHUNK, OCHUNK)], osem))
            for wo in writeouts[-2:]:
                wo.wait()


def _sc_gather(ptT, ntT, xT):
    mesh = plsc.VectorSubcoreMesh(core_axis_name="c", subcore_axis_name="s")
    f = pl.kernel(
        _sc_gather_body,
        out_type=jax.ShapeDtypeStruct((2 * HID, BATCH), jnp.float32),
        mesh=mesh,
        compiler_params=pltpu.CompilerParams(needs_layout_passes=False),
        scratch_types=[
            pltpu.VMEM((BATCH,), jnp.int32),
            pltpu.VMEM((VOC,), jnp.float32),
            pltpu.VMEM((OCHUNK,), jnp.float32),
            pltpu.VMEM((OCHUNK,), jnp.float32),
            pltpu.SemaphoreType.DMA,
        ],
    )
    return f(ptT, ntT, xT)


def _mlp_body(hT_ref, w1T_ref, b1_ref, w2_ref, b2_ref, out_ref):
    zT = jnp.dot(w1T_ref[...], hT_ref[...], preferred_element_type=jnp.float32)
    zT = jnp.maximum(zT + b1_ref[...], 0.0)
    out_ref[...] = jnp.sum(zT * w2_ref[...], axis=0, keepdims=True) + b2_ref[...]


def _mlp(hT, w1T, b1col, w2col, b2, block_cols=2048):
    grid = (BATCH // block_cols,)
    return pl.pallas_call(
        _mlp_body,
        grid=grid,
        in_specs=[
            pl.BlockSpec((2 * HID, block_cols), lambda i: (0, i)),
            pl.BlockSpec((16, 2 * HID), lambda i: (0, 0)),
            pl.BlockSpec((16, 1), lambda i: (0, 0)),
            pl.BlockSpec((16, 1), lambda i: (0, 0)),
            pl.BlockSpec((1, 1), lambda i: (0, 0)),
        ],
        out_specs=pl.BlockSpec((1, block_cols), lambda i: (0, i)),
        out_shape=jax.ShapeDtypeStruct((1, BATCH), jnp.float32),
    )(hT, w1T, b1col, w2col, b2)


@jax.jit
def kernel(x, emb_proton, emb_neutron, W1, b1, W2, b2):
    hT = _sc_gather(emb_proton.T, emb_neutron.T, x.T)
    outT = _mlp(hT, W1.T, b1.reshape(16, 1), W2, b2.reshape(1, 1))
    return outT.reshape(BATCH, 1)


# speedup vs baseline: 2.7666x; 1.2231x over previous
import jax
import jax.numpy as jnp
from jax import lax
from jax.experimental import pallas as pl
from jax.experimental.pallas import tpu as pltpu
from jax.experimental.pallas import tpu_sc as plsc

BATCH = 16384
HID = 64
NC = 2
NS = 16
DPW = HID // NS
VOC = 100000
OCHUNK = 4096
NOC = BATCH // OCHUNK


def _sc_gather_body(ptT_hbm, ntT_hbm, xT_hbm, hT_hbm,
                    idxv, rowv, ob0, ob1, osem):
    cid = lax.axis_index("c")
    sid = lax.axis_index("s")
    wid = sid * NC + cid
    obs = (ob0, ob1)

    for phase, tbl in ((0, ptT_hbm), (1, ntT_hbm)):
        pltpu.sync_copy(xT_hbm.at[phase], idxv)
        for k in range(2):
            d = wid * 2 + k
            pltpu.sync_copy(tbl.at[d], rowv)
            g = phase * HID + d
            writeouts = []
            for ci in range(NOC):
                ob = obs[ci % 2]
                if ci >= 2:
                    writeouts[ci - 2].wait()

                @pl.loop(0, OCHUNK // 128)
                def _gather(t, ci=ci, ob=ob):
                    for u in range(8):
                        iv = idxv[pl.ds(ci * OCHUNK + t * 128 + u * 16, 16)]
                        ob[pl.ds(t * 128 + u * 16, 16)] = (
                            plsc.load_gather(rowv, [iv]))

                writeouts.append(pltpu.async_copy(
                    ob, hT_hbm.at[g, pl.ds(ci * OCHUNK, OCHUNK)], osem))
            for wo in writeouts[-2:]:
                wo.wait()


def _sc_gather(ptT, ntT, xT):
    mesh = plsc.VectorSubcoreMesh(core_axis_name="c", subcore_axis_name="s")
    f = pl.kernel(
        _sc_gather_body,
        out_type=jax.ShapeDtypeStruct((2 * HID, BATCH), jnp.float32),
        mesh=mesh,
        compiler_params=pltpu.CompilerParams(needs_layout_passes=False),
        scratch_types=[
            pltpu.VMEM((BATCH,), jnp.int32),
            pltpu.VMEM((VOC,), jnp.float32),
            pltpu.VMEM((OCHUNK,), jnp.float32),
            pltpu.VMEM((OCHUNK,), jnp.float32),
            pltpu.SemaphoreType.DMA,
        ],
    )
    return f(ptT, ntT, xT)


def _mlp_body(hT_ref, w1T_ref, b1_ref, w2_ref, b2_ref, out_ref):
    zT = jnp.dot(w1T_ref[...], hT_ref[...], preferred_element_type=jnp.float32)
    zT = jnp.maximum(zT + b1_ref[...], 0.0)
    out_ref[...] = jnp.sum(zT * w2_ref[...], axis=0, keepdims=True) + b2_ref[...]


def _mlp(hT, w1T, b1col, w2col, b2, block_cols=2048):
    grid = (BATCH // block_cols,)
    return pl.pallas_call(
        _mlp_body,
        grid=grid,
        in_specs=[
            pl.BlockSpec((2 * HID, block_cols), lambda i: (0, i)),
            pl.BlockSpec((16, 2 * HID), lambda i: (0, 0)),
            pl.BlockSpec((16, 1), lambda i: (0, 0)),
            pl.BlockSpec((16, 1), lambda i: (0, 0)),
            pl.BlockSpec((1, 1), lambda i: (0, 0)),
        ],
        out_specs=pl.BlockSpec((1, block_cols), lambda i: (0, i)),
        out_shape=jax.ShapeDtypeStruct((1, BATCH), jnp.float32),
    )(hT, w1T, b1col, w2col, b2)


@jax.jit
def kernel(x, emb_proton, emb_neutron, W1, b1, W2, b2):
    hT = _sc_gather(emb_proton.T, emb_neutron.T, x.T)
    outT = _mlp(hT, W1.T, b1.reshape(16, 1), W2, b2.reshape(1, 1))
    return outT.reshape(BATCH, 1)
